# Initial kernel scaffold; baseline (speedup 1.0000x reference)
import functools

import jax
import jax.numpy as jnp
from jax import lax
from jax.experimental import pallas as pl
from jax.experimental.pallas import tpu as pltpu
from jax.experimental.pallas import tpu_sc as plsc

_NC, _NS = 2, 16
_NW = _NC * _NS
_LEAKY = 0.2
_GBLK = 128


@functools.lru_cache(maxsize=None)
def _sc_gather(R, C):
    rpw = R // _NW
    nblk = rpw // _GBLK
    assert rpw % _GBLK == 0 and R % _NW == 0
    mesh = plsc.VectorSubcoreMesh(core_axis_name="c", subcore_axis_name="s")

    @functools.partial(
        pl.kernel,
        mesh=mesh,
        out_type=jax.ShapeDtypeStruct((R, C), jnp.float32),
        scratch_types=[
            pltpu.VMEM((_GBLK,), jnp.int32),
            pltpu.VMEM((_GBLK, C), jnp.float32),
            pltpu.SemaphoreType.DMA,
        ],
    )
    def k(table_hbm, idx_hbm, out_hbm, idx_v, rows_v, sem):
        wid = lax.axis_index("s") * _NC + lax.axis_index("c")
        base0 = wid * rpw
        for b in range(nblk):
            base = base0 + b * _GBLK
            pltpu.sync_copy(idx_hbm.at[pl.ds(base, _GBLK)], idx_v)
            pltpu.async_copy(table_hbm.at[idx_v], rows_v, sem).wait()
            pltpu.sync_copy(rows_v, out_hbm.at[pl.ds(base, _GBLK)])

    return k


def _mesh_conv_tc(xT, graw, W5, bias, act_norm, res=None):
    Bb, E, C = xT.shape
    O = W5.shape[-1]

    def body(x_ref, g_ref, w_ref, b_ref, *rest):
        if res is not None:
            r_ref, o_ref = rest
        else:
            (o_ref,) = rest
        f1 = g_ref[0, :, 0, :]
        f2 = g_ref[0, :, 1, :]
        f3 = g_ref[0, :, 2, :]
        f4 = g_ref[0, :, 3, :]
        acc = jnp.dot(x_ref[0], w_ref[0], preferred_element_type=jnp.float32)
        acc += jnp.dot(f1 + f3, w_ref[1], preferred_element_type=jnp.float32)
        acc += jnp.dot(f2 + f4, w_ref[2], preferred_element_type=jnp.float32)
        acc += jnp.dot(jnp.abs(f1 - f3), w_ref[3],
                       preferred_element_type=jnp.float32)
        acc += jnp.dot(jnp.abs(f2 - f4), w_ref[4],
                       preferred_element_type=jnp.float32)
        acc = acc + b_ref[:, :]
        if act_norm:
            acc = jnp.where(acc >= 0, acc, _LEAKY * acc)
            m = jnp.mean(acc, axis=0, keepdims=True)
            v = jnp.mean((acc - m) * (acc - m), axis=0, keepdims=True)
            acc = (acc - m) * lax.rsqrt(v + 1e-5)
        if res is not None:
            acc = acc + r_ref[0]
        o_ref[0] = acc

    in_specs = [
        pl.BlockSpec((1, E, C), lambda b: (b, 0, 0)),
        pl.BlockSpec((1, E, 4, C), lambda b: (b, 0, 0, 0)),
        pl.BlockSpec((5, C, O), lambda b: (0, 0, 0)),
        pl.BlockSpec((1, O), lambda b: (0, 0)),
    ]
    args = [xT, graw, W5, bias]
    if res is not None:
        in_specs.append(pl.BlockSpec((1, E, O), lambda b: (b, 0, 0)))
        args.append(res)
    return pl.pallas_call(
        body,
        grid=(Bb,),
        in_specs=in_specs,
        out_specs=pl.BlockSpec((1, E, O), lambda b: (b, 0, 0)),
        out_shape=jax.ShapeDtypeStruct((Bb, E, O), jnp.float32),
    )(*args)


def _unpool_tc(groups, h1, tile=1024):
    Bb, E0, E1 = groups.shape
    O = h1.shape[-1]

    def body(g_ref, h_ref, o_ref):
        o_ref[0] = lax.dot_general(
            g_ref[0], h_ref[0], (((0,), (0,)), ((), ())),
            preferred_element_type=jnp.float32)

    return pl.pallas_call(
        body,
        grid=(Bb, E1 // tile),
        in_specs=[
            pl.BlockSpec((1, E0, tile), lambda b, j: (b, 0, j)),
            pl.BlockSpec((1, E0, O), lambda b, j: (b, 0, 0)),
        ],
        out_specs=pl.BlockSpec((1, tile, O), lambda b, j: (b, j, 0)),
        out_shape=jax.ShapeDtypeStruct((Bb, E1, O), jnp.float32),
    )(groups, h1)


def _w5(W):
    return W[:, :, 0, :].transpose(2, 1, 0)


def kernel(x, gemm0, gemm1, groups, nopool,
           W1, b1, W2, b2, W3, b3, Wf1, bf1, Wf2, bf2, Wf3, bf3):
    Bb, C_in, E0 = x.shape
    E1 = gemm1.shape[1]

    xT = x.transpose(0, 2, 1)
    nopoolT = nopool.transpose(0, 2, 1)

    offs0 = (jnp.arange(Bb, dtype=gemm0.dtype) * E0)[:, None, None]
    idx0 = (gemm0 + offs0).reshape(-1)
    offs1 = (jnp.arange(Bb, dtype=gemm1.dtype) * E1)[:, None, None]
    idx1 = (gemm1 + offs1).reshape(-1)

    def mc(h, idx, W, b, act_norm, res=None):
        _, E, C = h.shape
        R = Bb * E * 4
        g = _sc_gather(R, C)(h.reshape(Bb * E, C), idx)
        return _mesh_conv_tc(h, g.reshape(Bb, E, 4, C), _w5(W),
                             b.reshape(1, -1), act_norm, res)

    h1 = mc(xT, idx0, W1, b1, False)
    u = _unpool_tc(groups, h1)
    y2 = jnp.concatenate([u, nopoolT], axis=2)
    h2 = mc(y2, idx1, W2, b2, True)
    h3 = mc(h2, idx1, W3, b3, True, res=h2)
    h4 = mc(h3, idx1, Wf1, bf1, False)
    h5 = mc(h4, idx1, Wf2, bf2, True)
    h6 = mc(h5, idx1, Wf3, bf3, True, res=h5)
    return h6.transpose(0, 2, 1)


# speedup vs baseline: 7.9521x; 7.9521x over previous
import functools

import jax
import jax.numpy as jnp
from jax import lax
from jax.experimental import pallas as pl
from jax.experimental.pallas import tpu as pltpu
from jax.experimental.pallas import tpu_sc as plsc

_NC, _NS = 2, 16
_NW = _NC * _NS
_LEAKY = 0.2
_GBLK = 128


@functools.lru_cache(maxsize=None)
def _sc_gather(R, C):
    rpw = R // _NW
    nblk = rpw // _GBLK
    assert rpw % _GBLK == 0 and R % _NW == 0
    mesh = plsc.VectorSubcoreMesh(core_axis_name="c", subcore_axis_name="s")

    @functools.partial(
        pl.kernel,
        mesh=mesh,
        out_type=jax.ShapeDtypeStruct((R, C), jnp.float32),
        scratch_types=[
            pltpu.VMEM((_GBLK,), jnp.int32),
            pltpu.VMEM((_GBLK, C), jnp.float32),
            pltpu.SemaphoreType.DMA,
        ],
    )
    def k(table_hbm, idx_hbm, out_hbm, idx_v, rows_v, sem):
        wid = lax.axis_index("s") * _NC + lax.axis_index("c")
        base0 = wid * rpw
        for b in range(nblk):
            base = base0 + b * _GBLK
            pltpu.sync_copy(idx_hbm.at[pl.ds(base, _GBLK)], idx_v)
            pltpu.async_copy(table_hbm.at[idx_v], rows_v, sem).wait()
            pltpu.sync_copy(rows_v, out_hbm.at[pl.ds(base, _GBLK)])

    return k


def _mesh_conv_tc(xT, graw, W5, bias, act_norm, res=None):
    Bb, E, C = xT.shape
    O = W5.shape[-1]

    def body(x_ref, g_ref, w_ref, b_ref, *rest):
        if res is not None:
            r_ref, o_ref = rest
        else:
            (o_ref,) = rest
        f1 = g_ref[0, :, 0, :]
        f2 = g_ref[0, :, 1, :]
        f3 = g_ref[0, :, 2, :]
        f4 = g_ref[0, :, 3, :]
        acc = jnp.dot(x_ref[0], w_ref[0], preferred_element_type=jnp.float32)
        acc += jnp.dot(f1 + f3, w_ref[1], preferred_element_type=jnp.float32)
        acc += jnp.dot(f2 + f4, w_ref[2], preferred_element_type=jnp.float32)
        acc += jnp.dot(jnp.abs(f1 - f3), w_ref[3],
                       preferred_element_type=jnp.float32)
        acc += jnp.dot(jnp.abs(f2 - f4), w_ref[4],
                       preferred_element_type=jnp.float32)
        acc = acc + b_ref[:, :]
        if act_norm:
            acc = jnp.where(acc >= 0, acc, _LEAKY * acc)
            m = jnp.mean(acc, axis=0, keepdims=True)
            v = jnp.mean((acc - m) * (acc - m), axis=0, keepdims=True)
            acc = (acc - m) * lax.rsqrt(v + 1e-5)
        if res is not None:
            acc = acc + r_ref[0]
        o_ref[0] = acc

    in_specs = [
        pl.BlockSpec((1, E, C), lambda b: (b, 0, 0)),
        pl.BlockSpec((1, E, 4, C), lambda b: (b, 0, 0, 0)),
        pl.BlockSpec((5, C, O), lambda b: (0, 0, 0)),
        pl.BlockSpec((1, O), lambda b: (0, 0)),
    ]
    args = [xT, graw, W5, bias]
    if res is not None:
        in_specs.append(pl.BlockSpec((1, E, O), lambda b: (b, 0, 0)))
        args.append(res)
    return pl.pallas_call(
        body,
        grid=(Bb,),
        in_specs=in_specs,
        out_specs=pl.BlockSpec((1, E, O), lambda b: (b, 0, 0)),
        out_shape=jax.ShapeDtypeStruct((Bb, E, O), jnp.float32),
    )(*args)


def _unpool_tc(groups, h1, tile=1024):
    Bb, E0, E1 = groups.shape
    O = h1.shape[-1]

    def body(g_ref, h_ref, o_ref):
        o_ref[0] = lax.dot_general(
            g_ref[0], h_ref[0], (((0,), (0,)), ((), ())),
            preferred_element_type=jnp.float32)

    return pl.pallas_call(
        body,
        grid=(Bb, E1 // tile),
        in_specs=[
            pl.BlockSpec((1, E0, tile), lambda b, j: (b, 0, j)),
            pl.BlockSpec((1, E0, O), lambda b, j: (b, 0, 0)),
        ],
        out_specs=pl.BlockSpec((1, tile, O), lambda b, j: (b, j, 0)),
        out_shape=jax.ShapeDtypeStruct((Bb, E1, O), jnp.float32),
    )(groups, h1)


def _w5(W):
    return W[:, :, 0, :].transpose(2, 1, 0)


def kernel(x, gemm0, gemm1, groups, nopool,
           W1, b1, W2, b2, W3, b3, Wf1, bf1, Wf2, bf2, Wf3, bf3):
    Bb, C_in, E0 = x.shape
    E1 = gemm1.shape[1]

    xT = x.transpose(0, 2, 1)
    nopoolT = nopool.transpose(0, 2, 1)

    offs0 = (jnp.arange(Bb, dtype=gemm0.dtype) * E0)[:, None, None]
    idx0 = (gemm0 + offs0).reshape(-1)
    offs1 = (jnp.arange(Bb, dtype=gemm1.dtype) * E1)[:, None, None]
    idx1 = (gemm1 + offs1).reshape(-1)

    def mc(h, idx, W5, b, act_norm, res=None):
        _, E, C = h.shape
        R = Bb * E * 4
        g = _sc_gather(R, C)(h.reshape(Bb * E, C), idx)
        return _mesh_conv_tc(h, g.reshape(Bb, E, 4, C), W5,
                             b.reshape(1, -1), act_norm, res)

    W5f1 = jnp.pad(_w5(Wf1), ((0, 0), (0, 0), (0, 64)))
    W5f2 = jnp.pad(_w5(Wf2), ((0, 0), (0, 64), (0, 64)))
    W5f3 = jnp.pad(_w5(Wf3), ((0, 0), (0, 64), (0, 64)))
    bf1p = jnp.pad(bf1, (0, 64))
    bf2p = jnp.pad(bf2, (0, 64))
    bf3p = jnp.pad(bf3, (0, 64))

    h1 = mc(xT, idx0, _w5(W1), b1, False)
    u = _unpool_tc(groups, h1)
    y2 = jnp.concatenate([u, nopoolT], axis=2)
    h2 = mc(y2, idx1, _w5(W2), b2, True)
    h3 = mc(h2, idx1, _w5(W3), b3, True, res=h2)
    h4 = mc(h3, idx1, W5f1, bf1p, False)
    h5 = mc(h4, idx1, W5f2, bf2p, True)
    h6 = mc(h5, idx1, W5f3, bf3p, True, res=h5)
    return h6.transpose(0, 2, 1)[:, :64, :]


# speedup vs baseline: 8.8244x; 1.1097x over previous
import functools

import jax
import jax.numpy as jnp
from jax import lax
from jax.experimental import pallas as pl
from jax.experimental.pallas import tpu as pltpu
from jax.experimental.pallas import tpu_sc as plsc

_NC, _NS = 2, 16
_NW = _NC * _NS
_LEAKY = 0.2
_GBLK = 128


@functools.lru_cache(maxsize=None)
def _sc_gather(R, C):
    rpw = R // _NW
    nblk = rpw // _GBLK
    assert rpw % _GBLK == 0 and R % _NW == 0
    nbuf = min(3 if C >= 256 else 4, nblk)
    mesh = plsc.VectorSubcoreMesh(core_axis_name="c", subcore_axis_name="s")

    @functools.partial(
        pl.kernel,
        mesh=mesh,
        out_type=jax.ShapeDtypeStruct((R, C), jnp.float32),
        scratch_types=[
            pltpu.VMEM((nblk, _GBLK), jnp.int32),
            pltpu.VMEM((nbuf, _GBLK, C), jnp.float32),
            pltpu.SemaphoreType.DMA,
            pltpu.SemaphoreType.DMA,
        ],
    )
    def k(table_hbm, idx_hbm, out_hbm, idx_v, rows_v, gsem, wsem):
        wid = lax.axis_index("s") * _NC + lax.axis_index("c")
        base0 = wid * rpw
        pltpu.sync_copy(idx_hbm.at[pl.ds(wid * nblk, nblk)], idx_v)
        gathers = {}
        writes = {}
        for b in range(min(nbuf, nblk)):
            gathers[b] = pltpu.async_copy(
                table_hbm.at[idx_v.at[b]], rows_v.at[b % nbuf], gsem)
        for b in range(nblk):
            gathers[b].wait()
            writes[b] = pltpu.async_copy(
                rows_v.at[b % nbuf],
                out_hbm.at[pl.ds(base0 + b * _GBLK, _GBLK)], wsem)
            nb = b + nbuf
            if nb < nblk:
                writes[b].wait()
                gathers[nb] = pltpu.async_copy(
                    table_hbm.at[idx_v.at[nb]], rows_v.at[nb % nbuf], gsem)
        for b in range(max(nblk - nbuf, 0), nblk):
            writes[b].wait()

    return k


def _mesh_conv_tc(xT, graw, W5, bias, act_norm, res=None):
    Bb, E, C = xT.shape
    O = W5.shape[-1]

    def body(x_ref, g_ref, w_ref, b_ref, *rest):
        if res is not None:
            r_ref, o_ref = rest
        else:
            (o_ref,) = rest
        f1 = g_ref[0, :, 0, :]
        f2 = g_ref[0, :, 1, :]
        f3 = g_ref[0, :, 2, :]
        f4 = g_ref[0, :, 3, :]
        acc = jnp.dot(x_ref[0], w_ref[0], preferred_element_type=jnp.float32)
        acc += jnp.dot(f1 + f3, w_ref[1], preferred_element_type=jnp.float32)
        acc += jnp.dot(f2 + f4, w_ref[2], preferred_element_type=jnp.float32)
        acc += jnp.dot(jnp.abs(f1 - f3), w_ref[3],
                       preferred_element_type=jnp.float32)
        acc += jnp.dot(jnp.abs(f2 - f4), w_ref[4],
                       preferred_element_type=jnp.float32)
        acc = acc + b_ref[:, :]
        if act_norm:
            acc = jnp.where(acc >= 0, acc, _LEAKY * acc)
            m = jnp.mean(acc, axis=0, keepdims=True)
            v = jnp.mean((acc - m) * (acc - m), axis=0, keepdims=True)
            acc = (acc - m) * lax.rsqrt(v + 1e-5)
        if res is not None:
            acc = acc + r_ref[0]
        o_ref[0] = acc

    in_specs = [
        pl.BlockSpec((1, E, C), lambda b: (b, 0, 0)),
        pl.BlockSpec((1, E, 4, C), lambda b: (b, 0, 0, 0)),
        pl.BlockSpec((5, C, O), lambda b: (0, 0, 0)),
        pl.BlockSpec((1, O), lambda b: (0, 0)),
    ]
    args = [xT, graw, W5, bias]
    if res is not None:
        in_specs.append(pl.BlockSpec((1, E, O), lambda b: (b, 0, 0)))
        args.append(res)
    return pl.pallas_call(
        body,
        grid=(Bb,),
        in_specs=in_specs,
        out_specs=pl.BlockSpec((1, E, O), lambda b: (b, 0, 0)),
        out_shape=jax.ShapeDtypeStruct((Bb, E, O), jnp.float32),
    )(*args)


def _unpool_tc(groups, h1, tile=1024):
    Bb, E0, E1 = groups.shape
    O = h1.shape[-1]

    def body(g_ref, h_ref, o_ref):
        o_ref[0] = lax.dot_general(
            g_ref[0], h_ref[0], (((0,), (0,)), ((), ())),
            preferred_element_type=jnp.float32)

    return pl.pallas_call(
        body,
        grid=(Bb, E1 // tile),
        in_specs=[
            pl.BlockSpec((1, E0, tile), lambda b, j: (b, 0, j)),
            pl.BlockSpec((1, E0, O), lambda b, j: (b, 0, 0)),
        ],
        out_specs=pl.BlockSpec((1, tile, O), lambda b, j: (b, j, 0)),
        out_shape=jax.ShapeDtypeStruct((Bb, E1, O), jnp.float32),
    )(groups, h1)


def _w5(W):
    return W[:, :, 0, :].transpose(2, 1, 0)


def kernel(x, gemm0, gemm1, groups, nopool,
           W1, b1, W2, b2, W3, b3, Wf1, bf1, Wf2, bf2, Wf3, bf3):
    Bb, C_in, E0 = x.shape
    E1 = gemm1.shape[1]

    xT = x.transpose(0, 2, 1)
    nopoolT = nopool.transpose(0, 2, 1)

    offs0 = (jnp.arange(Bb, dtype=gemm0.dtype) * E0)[:, None, None]
    idx0 = (gemm0 + offs0).reshape(-1, _GBLK)
    offs1 = (jnp.arange(Bb, dtype=gemm1.dtype) * E1)[:, None, None]
    idx1 = (gemm1 + offs1).reshape(-1, _GBLK)

    def mc(h, idx, W5, b, act_norm, res=None):
        _, E, C = h.shape
        R = Bb * E * 4
        g = _sc_gather(R, C)(h.reshape(Bb * E, C), idx)
        return _mesh_conv_tc(h, g.reshape(Bb, E, 4, C), W5,
                             b.reshape(1, -1), act_norm, res)

    W5f1 = jnp.pad(_w5(Wf1), ((0, 0), (0, 0), (0, 64)))
    W5f2 = jnp.pad(_w5(Wf2), ((0, 0), (0, 64), (0, 64)))
    W5f3 = jnp.pad(_w5(Wf3), ((0, 0), (0, 64), (0, 64)))
    bf1p = jnp.pad(bf1, (0, 64))
    bf2p = jnp.pad(bf2, (0, 64))
    bf3p = jnp.pad(bf3, (0, 64))

    h1 = mc(xT, idx0, _w5(W1), b1, False)
    u = _unpool_tc(groups, h1)
    y2 = jnp.concatenate([u, nopoolT], axis=2)
    h2 = mc(y2, idx1, _w5(W2), b2, True)
    h3 = mc(h2, idx1, _w5(W3), b3, True, res=h2)
    h4 = mc(h3, idx1, W5f1, bf1p, False)
    h5 = mc(h4, idx1, W5f2, bf2p, True)
    h6 = mc(h5, idx1, W5f3, bf3p, True, res=h5)
    return h6.transpose(0, 2, 1)[:, :64, :]
